# baseline (speedup 1.0000x reference)
import functools

import jax
import jax.numpy as jnp
from jax import lax
from jax.experimental import pallas as pl
from jax.experimental.pallas import tpu as pltpu
from jax.experimental.pallas import tpu_sc as plsc

_D = 64
_B = 4096
_S = 200
_TOTAL = _B * _S
_NW = 32
_PER_W = _TOTAL // _NW
_CHUNK = 128
_NCH = _PER_W // _CHUNK
_NBUF = 8


def _emb_body(idx_hbm, table_hbm, out_hbm, idx_v, rows_v, sems):
    nc = plsc.get_sparse_core_info().num_cores
    wid = lax.axis_index("s") * nc + lax.axis_index("c")
    base = wid * _PER_W

    pltpu.sync_copy(idx_hbm.at[wid], idx_v)

    for b in range(_NBUF):
        pltpu.async_copy(table_hbm.at[idx_v.at[b]], rows_v.at[b], sems.at[b])

    def outer(i, carry):
        j0 = i * _NBUF
        for b in range(_NBUF):
            j = j0 + b
            pltpu.make_async_copy(
                table_hbm.at[pl.ds(0, _CHUNK)], rows_v.at[b], sems.at[b]
            ).wait()
            pltpu.sync_copy(
                rows_v.at[b], out_hbm.at[pl.ds(base + j * _CHUNK, _CHUNK)]
            )
            nj = j + _NBUF

            @pl.when(nj < _NCH)
            def _():
                pltpu.async_copy(
                    table_hbm.at[idx_v.at[nj]], rows_v.at[b], sems.at[b]
                )

        return carry

    lax.fori_loop(0, _NCH // _NBUF, outer, 0)


_emb = functools.partial(
    pl.kernel,
    mesh=plsc.VectorSubcoreMesh(core_axis_name="c", subcore_axis_name="s"),
    out_type=jax.ShapeDtypeStruct((_TOTAL, _D), jnp.float32),
    scratch_types=[
        pltpu.VMEM((_NCH, _CHUNK), jnp.int32),
        pltpu.VMEM((_NBUF, _CHUNK, _D), jnp.float32),
        pltpu.SemaphoreType.DMA((_NBUF,)),
    ],
    compiler_params=pltpu.CompilerParams(use_tc_tiling_on_sc=False),
)(_emb_body)


def kernel(token_ids, embedding_matrix):
    idx = token_ids.astype(jnp.int32).reshape(_NW, _NCH, _CHUNK)
    out = _emb(idx, embedding_matrix)
    return out.reshape(_B, _S, _D)
